# baseline (speedup 1.0000x reference)
import functools

import jax
import jax.numpy as jnp
from jax import lax
from jax.experimental import pallas as pl
from jax.experimental.pallas import tpu as pltpu
from jax.experimental.pallas import tpu_sc as plsc

_D = 1024
_SEQ = 2048
_NC = 2
_NS = 16
_NL = 16
_NW = _NC * _NS
_CH = 8
_NBUF = 4
_NSLICE = 4


def _sc_embed(ids_flat, word_table, pos_table):
    tok = ids_flat.shape[0]
    bpw = tok // _NW
    nchunk = bpw // _CH
    mesh = plsc.VectorSubcoreMesh(core_axis_name="c", subcore_axis_name="s")

    @functools.partial(
        pl.kernel,
        mesh=mesh,
        out_type=jax.ShapeDtypeStruct((tok, _D), jnp.float32),
        scratch_types=[
            pltpu.VMEM((bpw,), jnp.int32),
            pltpu.VMEM((_NBUF, _CH, _D), jnp.float32),
            pltpu.VMEM((2, _CH, _D), jnp.float32),
            pltpu.VMEM_SHARED((_NSLICE * bpw, _D), jnp.float32),
        ]
        + [pltpu.SemaphoreType.DMA] * (2 + 2 * _NBUF),
    )
    def k(ids_hbm, wt_hbm, pt_hbm, out_hbm, idx_v, rows, pos, pos_sh, *sems):
        p2 = sems[0:2]
        gs = sems[2:2 + _NBUF]
        ws = sems[2 + _NBUF:2 + 2 * _NBUF]
        cid = lax.axis_index("c")
        sid = lax.axis_index("s")
        wid = sid * _NC + cid
        base = wid * bpw
        sh_base = lax.rem(sid, _NSLICE) * bpw

        _worker_body(ids_hbm, wt_hbm, pt_hbm, out_hbm, idx_v, rows, pos,
                     pos_sh, p2, gs, ws, cid, sid, base, sh_base,
                     bpw, nchunk)

    def _worker_body(ids_hbm, wt_hbm, pt_hbm, out_hbm, idx_v, rows, pos,
                     pos_sh, p2, gs, ws, cid, sid, base, sh_base,
                     bpw, nchunk):
        quarter = bpw // _NSLICE
        k_slice = lax.div(sid, _NSLICE)
        src = (lax.rem(2 * k_slice + cid, 2 * _NSLICE) * bpw
               + lax.rem(sid, _NSLICE) * quarter)
        stage = pltpu.async_copy(
            pt_hbm.at[pl.ds(src, quarter)],
            pos_sh.at[pl.ds(sid * quarter, quarter)], p2[0])

        pltpu.sync_copy(ids_hbm.at[pl.ds(base, bpw)], idx_v)

        def start_gather(c, b):
            pltpu.async_copy(
                wt_hbm.at[idx_v.at[pl.ds(c * _CH, _CH)]], rows.at[b], gs[b])

        for b in range(3):
            start_gather(b, b)

        stage.wait()
        plsc.subcore_barrier()

        def start_pos(c, pb_idx):
            pltpu.async_copy(
                pos_sh.at[pl.ds(sh_base + c * _CH, _CH)],
                pos.at[pb_idx], p2[pb_idx])

        for c in range(2):
            start_pos(c, c % 2)

        @pl.loop(0, nchunk, step=_NBUF)
        def _(c):
            for b in range(_NBUF):
                cc = c + b

                @pl.when(cc + 3 < nchunk)
                def _():
                    bn = (b + 3) % _NBUF

                    @pl.when(cc >= 1)
                    def _():
                        pltpu.make_async_copy(
                            wt_hbm.at[pl.ds(0, _CH)], rows.at[bn],
                            ws[bn]).wait()

                    start_gather(cc + 3, bn)

                pltpu.make_async_copy(
                    wt_hbm.at[pl.ds(0, _CH)], rows.at[b], gs[b]).wait()
                pltpu.make_async_copy(
                    pt_hbm.at[pl.ds(0, _CH)], pos.at[b % 2], p2[b % 2]).wait()

                rb = rows.at[b]
                pb = pos.at[b % 2]

                @pl.loop(0, _CH)
                def _(r):
                    for u in range(_D // _NL):
                        slc = (r, pl.ds(u * _NL, _NL))
                        rb[slc] += pb[slc]

                pltpu.async_copy(
                    rb, out_hbm.at[pl.ds(base + cc * _CH, _CH)], ws[b])

                @pl.when(cc + 2 < nchunk)
                def _():
                    start_pos(cc + 2, b % 2)

        for b in range(_NBUF):
            pltpu.make_async_copy(
                wt_hbm.at[pl.ds(0, _CH)], rows.at[b], ws[b]).wait()

    return k(ids_flat, word_table, pos_table)


def kernel(input_ids, word_table, pos_table):
    b, s = input_ids.shape
    ids_flat = input_ids.reshape(-1).astype(jnp.int32)
    out = _sc_embed(ids_flat, word_table, pos_table)
    return out.reshape(b, s, _D)


# speedup vs baseline: 1.1139x; 1.1139x over previous
import functools

import jax
import jax.numpy as jnp
from jax import lax
from jax.experimental import pallas as pl
from jax.experimental.pallas import tpu as pltpu
from jax.experimental.pallas import tpu_sc as plsc

_D = 1024
_SEQ = 2048
_NC = 2
_NS = 16
_NL = 16
_NW = _NC * _NS
_CH = 8
_NBUF = 4
_NSLICE = 4


def _sc_embed(ids_flat, word_table, pos_table):
    tok = ids_flat.shape[0]
    bpw = tok // _NW
    nchunk = bpw // _CH
    mesh = plsc.VectorSubcoreMesh(core_axis_name="c", subcore_axis_name="s")

    @functools.partial(
        pl.kernel,
        mesh=mesh,
        out_type=jax.ShapeDtypeStruct((tok, _D), jnp.float32),
        scratch_types=[
            pltpu.VMEM((bpw,), jnp.int32),
            pltpu.VMEM((_NBUF, _CH, _D), jnp.float32),
            pltpu.VMEM((2, _CH, _D), jnp.float32),
            pltpu.VMEM_SHARED((_NSLICE * bpw, _D), jnp.float32),
        ]
        + [pltpu.SemaphoreType.DMA] * (2 + 2 * _NBUF),
    )
    def k(ids_hbm, wt_hbm, pt_hbm, out_hbm, idx_v, rows, pos, pos_sh, *sems):
        p2 = sems[0:2]
        gs = sems[2:2 + _NBUF]
        ws = sems[2 + _NBUF:2 + 2 * _NBUF]
        cid = lax.axis_index("c")
        sid = lax.axis_index("s")
        wid = sid * _NC + cid
        base = wid * bpw
        sh_base = lax.rem(sid, _NSLICE) * bpw

        _worker_body(ids_hbm, wt_hbm, pt_hbm, out_hbm, idx_v, rows, pos,
                     pos_sh, p2, gs, ws, cid, sid, base, sh_base,
                     bpw, nchunk)

    def _worker_body(ids_hbm, wt_hbm, pt_hbm, out_hbm, idx_v, rows, pos,
                     pos_sh, p2, gs, ws, cid, sid, base, sh_base,
                     bpw, nchunk):
        quarter = bpw // _NSLICE
        k_slice = lax.div(sid, _NSLICE)
        src = (lax.rem(2 * k_slice + cid, 2 * _NSLICE) * bpw
               + lax.rem(sid, _NSLICE) * quarter)
        stage = pltpu.async_copy(
            pt_hbm.at[pl.ds(src, quarter)],
            pos_sh.at[pl.ds(sid * quarter, quarter)], p2[0])

        pltpu.sync_copy(ids_hbm.at[pl.ds(base, bpw)], idx_v)

        def start_gather(c, b):
            pltpu.async_copy(
                wt_hbm.at[idx_v.at[pl.ds(c * _CH, _CH)]], rows.at[b], gs[b])

        for b in range(2):
            start_gather(b, b)

        stage.wait()
        plsc.subcore_barrier()

        def start_pos(c, pb_idx):
            pltpu.async_copy(
                pos_sh.at[pl.ds(sh_base + c * _CH, _CH)],
                pos.at[pb_idx], p2[pb_idx])

        for c in range(2):
            start_pos(c, c % 2)

        @pl.loop(0, nchunk, step=_NBUF)
        def _(c):
            for b in range(_NBUF):
                cc = c + b

                @pl.when(cc + 2 < nchunk)
                def _():
                    bn = (b + 2) % _NBUF

                    @pl.when(cc >= 2)
                    def _():
                        pltpu.make_async_copy(
                            wt_hbm.at[pl.ds(0, _CH)], rows.at[bn],
                            ws[bn]).wait()

                    start_gather(cc + 2, bn)

                pltpu.make_async_copy(
                    wt_hbm.at[pl.ds(0, _CH)], rows.at[b], gs[b]).wait()
                pltpu.make_async_copy(
                    pt_hbm.at[pl.ds(0, _CH)], pos.at[b % 2], p2[b % 2]).wait()

                rb = rows.at[b]
                pb = pos.at[b % 2]

                @pl.loop(0, _CH)
                def _(r):
                    for u in range(_D // _NL):
                        slc = (r, pl.ds(u * _NL, _NL))
                        rb[slc] += pb[slc]

                pltpu.async_copy(
                    rb, out_hbm.at[pl.ds(base + cc * _CH, _CH)], ws[b])

                @pl.when(cc + 2 < nchunk)
                def _():
                    start_pos(cc + 2, b % 2)

        for b in range(_NBUF):
            pltpu.make_async_copy(
                wt_hbm.at[pl.ds(0, _CH)], rows.at[b], ws[b]).wait()

    return k(ids_flat, word_table, pos_table)


def kernel(input_ids, word_table, pos_table):
    b, s = input_ids.shape
    ids_flat = input_ids.reshape(-1).astype(jnp.int32)
    out = _sc_embed(ids_flat, word_table, pos_table)
    return out.reshape(b, s, _D)
